# baseline (speedup 1.0000x reference)
import jax
import jax.numpy as jnp
from jax import lax
from jax.experimental import pallas as pl
from jax.experimental.pallas import tpu as pltpu
from jax.experimental.pallas import tpu_sc as plsc

N_CH = 192
N_PTS = 7
BOUND = 2.7
N_REG = 6
ROW = 224 * 224
NROWS = 768
NW = 32
CPR = 4
CHUNK = ROW // CPR
LANES = 16
NCHUNKS = (NROWS // NW) * CPR

_INV_LEN = N_REG / (2.0 * BOUND)
_SHIFT = BOUND * _INV_LEN


def _take16(vec, idx):
  return vec.at[idx].get(mode="promise_in_bounds")


def _body(x_hbm, pts_hbm, out_hbm,
          pts_v, tin, tout, spm,
          h2s, s2t, t2s, s2h):
  sc = lax.axis_index("c")
  tile = lax.axis_index("s")
  wid = tile * 2 + sc
  base = wid * NCHUNKS

  pltpu.sync_copy(pts_hbm, pts_v)

  lanes = lax.iota(jnp.int32, LANES)
  shift_idx = jnp.minimum(lanes + 1, LANES - 1)
  lanes_f = lanes.astype(jnp.float32)


  def stage_a(i, par):
    pltpu.async_copy(x_hbm.at[pl.ds((base + i) * CHUNK, CHUNK)],
                     spm.at[tile, par], h2s[par])

  def stage_b(i, par):
    pltpu.make_async_copy(x_hbm.at[pl.ds(0, CHUNK)], spm.at[tile, par],
                          h2s[par]).wait()
    pltpu.async_copy(spm.at[tile, par], tin[par], s2t[par])

  def stage_c(i, par):
    pltpu.make_async_copy(spm.at[tile, par], tin[par], s2t[par]).wait()
    row = (base + i) // CPR
    ch = lax.rem(row, N_CH)
    p = pts_v[pl.ds(ch * LANES, LANES)]
    d = _take16(p, shift_idx) - p
    a = p - lanes_f * d

    @plsc.parallel_loop(0, CHUNK, step=LANES, unroll=16)
    def _(off):
      xv = tin[par][pl.ds(off, LANES)]
      xn = xv * _INV_LEN + _SHIFT
      ri = jnp.minimum(jnp.maximum(xn.astype(jnp.int32), 0), N_REG - 1)
      tout[par][pl.ds(off, LANES)] = _take16(a, ri) + xn * _take16(d, ri)

  def stage_d(i, par):
    @pl.when(i >= 2)
    def _():
      pltpu.make_async_copy(spm.at[tile, 2 + par],
                            out_hbm.at[pl.ds(0, CHUNK)], s2h[par]).wait()
    pltpu.async_copy(tout[par], spm.at[tile, 2 + par], t2s[par])

  def stage_e(i, par):
    pltpu.make_async_copy(tout[par], spm.at[tile, 2 + par], t2s[par]).wait()
    pltpu.async_copy(spm.at[tile, 2 + par],
                     out_hbm.at[pl.ds((base + i) * CHUNK, CHUNK)], s2h[par])


  def iteration(i, par):
    @pl.when(jnp.logical_and(i >= 4, i < NCHUNKS + 4))
    def _():
      stage_e(i - 4, par)

    @pl.when(jnp.logical_and(i >= 3, i < NCHUNKS + 3))
    def _():
      stage_d(i - 3, (par + 1) % 2)

    @pl.when(jnp.logical_and(i >= 2, i < NCHUNKS + 2))
    def _():
      stage_c(i - 2, par)

    @pl.when(jnp.logical_and(i >= 1, i < NCHUNKS + 1))
    def _():
      stage_b(i - 1, (par + 1) % 2)

    @pl.when(i < NCHUNKS)
    def _():
      stage_a(i, par)

  def outer(i2, carry):
    i = i2 * 2
    iteration(i, 0)
    iteration(i + 1, 1)
    return carry

  lax.fori_loop(0, (NCHUNKS + 4) // 2, outer, 0)
  for par in range(2):
    pltpu.make_async_copy(spm.at[tile, 2 + par],
                          out_hbm.at[pl.ds(0, CHUNK)], s2h[par]).wait()


@jax.jit
def _pwlu_sc(x_flat, pts_pad_flat):
  mesh = plsc.VectorSubcoreMesh(core_axis_name="c", subcore_axis_name="s")
  return pl.kernel(
      _body,
      out_type=jax.ShapeDtypeStruct((NROWS * ROW,), jnp.float32),
      mesh=mesh,
      scratch_types=[
          pltpu.VMEM((N_CH * LANES,), jnp.float32),
          [pltpu.VMEM((CHUNK,), jnp.float32) for _ in range(2)],
          [pltpu.VMEM((CHUNK,), jnp.float32) for _ in range(2)],
          pltpu.VMEM_SHARED((16, 4, CHUNK), jnp.float32),
          [pltpu.SemaphoreType.DMA for _ in range(2)],
          [pltpu.SemaphoreType.DMA for _ in range(2)],
          [pltpu.SemaphoreType.DMA for _ in range(2)],
          [pltpu.SemaphoreType.DMA for _ in range(2)],
      ],
  )(x_flat, pts_pad_flat)


def kernel(x, points):
  pts_pad = jnp.zeros((N_CH, LANES), jnp.float32).at[:, :N_PTS].set(points)
  out = _pwlu_sc(x.reshape(-1), pts_pad.reshape(-1))
  return out.reshape(x.shape)
